# baseline (device time: 411932 ns/iter reference)
import jax
import jax.numpy as jnp
from jax import lax
from jax.experimental import pallas as pl
from jax.experimental.pallas import tpu as pltpu


def kernel(x):
    m, n = x.shape
    K = 16
    mc = m // K

    def body(x_ref, out_ref, xb_ref, f32_bufs, bf_bufs, ld_sems, ow_sems,
             xb_sems, send_sems, recv_sems):
        my_x = lax.axis_index("x")
        my_y = lax.axis_index("y")
        my_z = lax.axis_index("z")
        nbr = (my_x, 1 - my_y, my_z)

        barrier = pltpu.get_barrier_semaphore()
        pl.semaphore_signal(
            barrier, inc=1, device_id=nbr, device_id_type=pl.DeviceIdType.MESH
        )
        pl.semaphore_wait(barrier, 1)

        base = my_y * m

        def load(c):
            cp = pltpu.make_async_copy(
                x_ref.at[pl.ds(c * mc, mc), :],
                f32_bufs.at[c % 2],
                ld_sems.at[c % 2],
            )
            cp.start()
            return cp

        loads = [None] * K
        rdmas = [None] * K
        owns = [None] * K
        loads[0] = load(0)
        for c in range(K):
            sl = c % 2
            if c + 1 < K:
                loads[c + 1] = load(c + 1)
            loads[c].wait()
            if c >= 2:
                owns[c - 2].wait()
            bf_bufs[sl, :, :] = f32_bufs[sl, :, :].astype(jnp.bfloat16)
            xb_st = pltpu.make_async_copy(
                bf_bufs.at[sl], xb_ref.at[pl.ds(c * mc, mc), :], xb_sems.at[sl]
            )
            xb_st.start()
            owns[c] = pltpu.make_async_copy(
                bf_bufs.at[sl],
                out_ref.at[pl.ds(base + c * mc, mc), :],
                ow_sems.at[sl],
            )
            owns[c].start()
            xb_st.wait()
            rdmas[c] = pltpu.make_async_remote_copy(
                src_ref=xb_ref.at[pl.ds(c * mc, mc), :],
                dst_ref=out_ref.at[pl.ds(base + c * mc, mc), :],
                send_sem=send_sems.at[c],
                recv_sem=recv_sems.at[c],
                device_id=nbr,
                device_id_type=pl.DeviceIdType.MESH,
            )
            rdmas[c].start()

        owns[K - 2].wait()
        owns[K - 1].wait()
        for c in range(K):
            rdmas[c].wait()

    out, _ = pl.pallas_call(
        body,
        out_shape=[
            jax.ShapeDtypeStruct((2 * m, n), jnp.bfloat16),
            jax.ShapeDtypeStruct((m, n), jnp.bfloat16),
        ],
        in_specs=[pl.BlockSpec(memory_space=pl.ANY)],
        out_specs=[
            pl.BlockSpec(memory_space=pl.ANY),
            pl.BlockSpec(memory_space=pl.ANY),
        ],
        scratch_shapes=[
            pltpu.VMEM((2, mc, n), jnp.float32),
            pltpu.VMEM((2, mc, n), jnp.bfloat16),
            pltpu.SemaphoreType.DMA((2,)),
            pltpu.SemaphoreType.DMA((2,)),
            pltpu.SemaphoreType.DMA((2,)),
            pltpu.SemaphoreType.DMA((K,)),
            pltpu.SemaphoreType.DMA((K,)),
        ],
        compiler_params=pltpu.CompilerParams(collective_id=0),
    )(x)
    return out
